# baseline (speedup 1.0000x reference)
import functools

import jax
import jax.numpy as jnp
import numpy as np
from jax import lax
from jax.experimental import pallas as pl
from jax.experimental.pallas import tpu as pltpu
from jax.experimental.pallas import tpu_sc as plsc

NTILES = 16
NCORES = 2
CHUNK = 80
NSLOT = 4


def _tc_matmul(x, w, *, n, d_in, d_out, dh, bm):
    def body(x_ref, w_ref, y0_ref, y1_ref):
        y = lax.dot_general(x_ref[...], w_ref[...],
                            (((1,), (1,)), ((), ())),
                            preferred_element_type=jnp.float32)
        y0_ref[...] = y[:, :dh]
        y1_ref[...] = y[:, dh:]

    return pl.pallas_call(
        body,
        grid=(n // bm,),
        in_specs=[
            pl.BlockSpec((bm, d_in), lambda i: (i, 0)),
            pl.BlockSpec((d_out, d_in), lambda i: (0, 0)),
        ],
        out_specs=[
            pl.BlockSpec((bm, dh), lambda i: (i, 0)),
            pl.BlockSpec((bm, dh), lambda i: (i, 0)),
        ],
        out_shape=[
            jax.ShapeDtypeStruct((n, dh), jnp.float32),
            jax.ShapeDtypeStruct((n, dh), jnp.float32),
        ],
    )(x, w)


def _sc_aggregate(y0, y1, eidx, binit, *, n, n_pad, dh, nch):
    mesh = plsc.VectorSubcoreMesh(core_axis_name="c", subcore_axis_name="s")
    rpt = n_pad // NTILES
    tail = n - (NTILES - 1) * rpt

    @functools.partial(
        pl.kernel,
        out_type=jax.ShapeDtypeStruct((n, NCORES * dh), jnp.float32),
        mesh=mesh,
        scratch_types=[
            pltpu.VMEM((NSLOT, CHUNK), jnp.int32),
            pltpu.VMEM((NSLOT, CHUNK), jnp.int32),
            pltpu.VMEM((NSLOT, CHUNK, dh), jnp.float32),
            pltpu.VMEM_SHARED((n_pad, dh), jnp.float32),
            [pltpu.SemaphoreType.DMA] * NSLOT,
            [pltpu.SemaphoreType.DMA] * NSLOT,
            [pltpu.SemaphoreType.DMA] * NSLOT,
        ],
    )
    def sc_kernel(y0h, y1h, eh, bh, out, srcx, dstx, rows,
                  agg, si, sd, sg):
        c = lax.axis_index("c")
        s = lax.axis_index("s")
        srch = eh.at[0]
        dsth = eh.at[1]

        pltpu.sync_copy(bh.at[c], agg.at[pl.ds(s * rpt, rpt)])
        plsc.subcore_barrier()

        def run(yh):
            for p in range(NSLOT):
                pltpu.async_copy(srch.at[s, p], srcx.at[p], si[p])
                pltpu.async_copy(dsth.at[s, p], dstx.at[p], sd[p])
            for p in range(NSLOT):
                pltpu.make_async_copy(srch.at[s, p], srcx.at[p], si[p]).wait()
                pltpu.async_copy(yh.at[srcx.at[p]], rows.at[p], sg[p])

            def step(k, carry):
                j0 = NSLOT * k
                for p in range(NSLOT):
                    j = j0 + p
                    pltpu.make_async_copy(
                        yh.at[srcx.at[p]], rows.at[p], sg[p]).wait()

                    @pl.when(j + NSLOT < nch)
                    def _():
                        pltpu.async_copy(
                            srch.at[s, j + NSLOT], srcx.at[p], si[p])

                    pltpu.make_async_copy(
                        dsth.at[s, j], dstx.at[p], sd[p]).wait()
                    pltpu.sync_copy(rows.at[p], agg.at[dstx.at[p]], add=True)

                    @pl.when(j + NSLOT < nch)
                    def _():
                        pltpu.async_copy(
                            dsth.at[s, j + NSLOT], dstx.at[p], sd[p])
                        pltpu.make_async_copy(
                            srch.at[s, j + NSLOT], srcx.at[p], si[p]).wait()
                        pltpu.async_copy(yh.at[srcx.at[p]], rows.at[p], sg[p])

                return carry

            lax.fori_loop(0, nch // NSLOT, step, 0)

        @pl.when(c == 0)
        def _():
            run(y0h)

        @pl.when(c == 1)
        def _():
            run(y1h)

        plsc.subcore_barrier()

        @pl.when(s < NTILES - 1)
        def _():
            pltpu.sync_copy(
                agg.at[pl.ds(s * rpt, rpt)],
                out.at[pl.ds(s * rpt, rpt), pl.ds(c * dh, dh)])

        @pl.when(s == NTILES - 1)
        def _():
            pltpu.sync_copy(
                agg.at[pl.ds((NTILES - 1) * rpt, tail)],
                out.at[pl.ds((NTILES - 1) * rpt, tail), pl.ds(c * dh, dh)])

    return sc_kernel(y0, y1, eidx, binit)


def kernel(X, edge_index, W, b):
    n, d_in = X.shape
    d_out = W.shape[0]
    e = edge_index.shape[1]
    dh = d_out // 2

    n_pad = ((n + 8 * NTILES - 1) // (8 * NTILES)) * (8 * NTILES)
    if n_pad == n:
        n_pad = n + 8 * NTILES
    nch = (e + NTILES * CHUNK - 1) // (NTILES * CHUNK)
    nch = ((nch + NSLOT - 1) // NSLOT) * NSLOT
    e_pad = NTILES * CHUNK * nch
    npad_e = e_pad - e

    ei = edge_index
    if npad_e:
        pad_src = ((np.arange(npad_e) * 37) % n).astype(np.int32)
        pad_dst = (n + (np.arange(npad_e) % (n_pad - n))).astype(np.int32)
        ei = jnp.concatenate(
            [ei, jnp.asarray(np.stack([pad_src, pad_dst]))], axis=1)
    eidx = ei.reshape(2, NTILES, nch, CHUNK)

    y0, y1 = _tc_matmul(X, W, n=n, d_in=d_in, d_out=d_out, dh=dh, bm=1000)

    rpt = n_pad // NTILES
    binit = jnp.broadcast_to(b.reshape(NCORES, 1, dh), (NCORES, rpt, dh))

    return _sc_aggregate(y0, y1, eidx, binit,
                         n=n, n_pad=n_pad, dh=dh, nch=nch)


# speedup vs baseline: 1.0323x; 1.0323x over previous
import functools

import jax
import jax.numpy as jnp
import numpy as np
from jax import lax
from jax.experimental import pallas as pl
from jax.experimental.pallas import tpu as pltpu
from jax.experimental.pallas import tpu_sc as plsc

NTILES = 16
NCORES = 2
CHUNK = 112
NSLOT = 3


def _tc_matmul(x, w, *, n, d_in, d_out, dh, bm):
    def body(x_ref, w_ref, y0_ref, y1_ref):
        y = lax.dot_general(x_ref[...], w_ref[...],
                            (((1,), (1,)), ((), ())),
                            preferred_element_type=jnp.float32)
        y0_ref[...] = y[:, :dh]
        y1_ref[...] = y[:, dh:]

    return pl.pallas_call(
        body,
        grid=(n // bm,),
        in_specs=[
            pl.BlockSpec((bm, d_in), lambda i: (i, 0)),
            pl.BlockSpec((d_out, d_in), lambda i: (0, 0)),
        ],
        out_specs=[
            pl.BlockSpec((bm, dh), lambda i: (i, 0)),
            pl.BlockSpec((bm, dh), lambda i: (i, 0)),
        ],
        out_shape=[
            jax.ShapeDtypeStruct((n, dh), jnp.float32),
            jax.ShapeDtypeStruct((n, dh), jnp.float32),
        ],
    )(x, w)


def _sc_aggregate(y0, y1, eidx, binit, *, n, n_pad, dh, nch):
    mesh = plsc.VectorSubcoreMesh(core_axis_name="c", subcore_axis_name="s")
    rpt = n_pad // NTILES
    tail = n - (NTILES - 1) * rpt

    @functools.partial(
        pl.kernel,
        out_type=jax.ShapeDtypeStruct((n, NCORES * dh), jnp.float32),
        mesh=mesh,
        scratch_types=[
            pltpu.VMEM((NSLOT, CHUNK), jnp.int32),
            pltpu.VMEM((NSLOT, CHUNK), jnp.int32),
            pltpu.VMEM((NSLOT, CHUNK, dh), jnp.float32),
            pltpu.VMEM_SHARED((n_pad, dh), jnp.float32),
            [pltpu.SemaphoreType.DMA] * NSLOT,
            [pltpu.SemaphoreType.DMA] * NSLOT,
            [pltpu.SemaphoreType.DMA] * NSLOT,
        ],
    )
    def sc_kernel(y0h, y1h, eh, bh, out, srcx, dstx, rows,
                  agg, si, sd, sg):
        c = lax.axis_index("c")
        s = lax.axis_index("s")
        srch = eh.at[0]
        dsth = eh.at[1]

        pltpu.sync_copy(bh.at[c], agg.at[pl.ds(s * rpt, rpt)])
        plsc.subcore_barrier()

        def run(yh):
            for p in range(NSLOT):
                pltpu.async_copy(srch.at[s, p], srcx.at[p], si[p])
                pltpu.async_copy(dsth.at[s, p], dstx.at[p], sd[p])
            for p in range(NSLOT):
                pltpu.make_async_copy(srch.at[s, p], srcx.at[p], si[p]).wait()
                pltpu.async_copy(yh.at[srcx.at[p]], rows.at[p], sg[p])

            def step(k, carry):
                j0 = NSLOT * k
                for p in range(NSLOT):
                    j = j0 + p
                    pltpu.make_async_copy(
                        yh.at[srcx.at[p]], rows.at[p], sg[p]).wait()

                    @pl.when(j + NSLOT < nch)
                    def _():
                        pltpu.async_copy(
                            srch.at[s, j + NSLOT], srcx.at[p], si[p])

                    pltpu.make_async_copy(
                        dsth.at[s, j], dstx.at[p], sd[p]).wait()
                    pltpu.sync_copy(rows.at[p], agg.at[dstx.at[p]], add=True)

                    @pl.when(j + NSLOT < nch)
                    def _():
                        pltpu.async_copy(
                            dsth.at[s, j + NSLOT], dstx.at[p], sd[p])
                        pltpu.make_async_copy(
                            srch.at[s, j + NSLOT], srcx.at[p], si[p]).wait()
                        pltpu.async_copy(yh.at[srcx.at[p]], rows.at[p], sg[p])

                return carry

            lax.fori_loop(0, nch // NSLOT, step, 0)

        @pl.when(c == 0)
        def _():
            run(y0h)

        @pl.when(c == 1)
        def _():
            run(y1h)

        plsc.subcore_barrier()

        @pl.when(s < NTILES - 1)
        def _():
            pltpu.sync_copy(
                agg.at[pl.ds(s * rpt, rpt)],
                out.at[pl.ds(s * rpt, rpt), pl.ds(c * dh, dh)])

        @pl.when(s == NTILES - 1)
        def _():
            pltpu.sync_copy(
                agg.at[pl.ds((NTILES - 1) * rpt, tail)],
                out.at[pl.ds((NTILES - 1) * rpt, tail), pl.ds(c * dh, dh)])

    return sc_kernel(y0, y1, eidx, binit)


def kernel(X, edge_index, W, b):
    n, d_in = X.shape
    d_out = W.shape[0]
    e = edge_index.shape[1]
    dh = d_out // 2

    n_pad = ((n + 8 * NTILES - 1) // (8 * NTILES)) * (8 * NTILES)
    if n_pad == n:
        n_pad = n + 8 * NTILES
    nch = (e + NTILES * CHUNK - 1) // (NTILES * CHUNK)
    nch = ((nch + NSLOT - 1) // NSLOT) * NSLOT
    e_pad = NTILES * CHUNK * nch
    npad_e = e_pad - e

    ei = edge_index
    if npad_e:
        pad_src = ((np.arange(npad_e) * 37) % n).astype(np.int32)
        pad_dst = (n + (np.arange(npad_e) % (n_pad - n))).astype(np.int32)
        ei = jnp.concatenate(
            [ei, jnp.asarray(np.stack([pad_src, pad_dst]))], axis=1)
    eidx = ei.reshape(2, NTILES, nch, CHUNK)

    y0, y1 = _tc_matmul(X, W, n=n, d_in=d_in, d_out=d_out, dh=dh, bm=2000)

    rpt = n_pad // NTILES
    binit = jnp.broadcast_to(b.reshape(NCORES, 1, dh), (NCORES, rpt, dh))

    return _sc_aggregate(y0, y1, eidx, binit,
                         n=n, n_pad=n_pad, dh=dh, nch=nch)
